# baseline (speedup 1.0000x reference)
import functools

import jax
import jax.numpy as jnp
from jax import lax
from jax.experimental import pallas as pl
from jax.experimental.pallas import tpu as pltpu
from jax.experimental.pallas import tpu_sc as plsc

NUM_CAT = 26
VOCAB = 100000
EMBED = 32
NUM_DENSE = 13
CROSS_LAYERS = 3
BATCH = 4096
INPUT_DIM = NUM_DENSE + NUM_CAT * EMBED
PAD_DIM = 896
NIDX = BATCH * NUM_CAT
CHUNK = 128
BB = 512


@functools.lru_cache(maxsize=1)
def _make_gather():
    info = plsc.get_sparse_core_info()
    nc, ns = info.num_cores, info.num_subcores
    nw = nc * ns
    b_per_w = NIDX // nw
    n_chunks = b_per_w // CHUNK
    mesh = plsc.VectorSubcoreMesh(core_axis_name="c", subcore_axis_name="s")

    @functools.partial(
        pl.kernel,
        mesh=mesh,
        compiler_params=pltpu.CompilerParams(use_tc_tiling_on_sc=False),
        out_type=jax.ShapeDtypeStruct((nw, b_per_w, EMBED), jnp.float32),
        scratch_types=[
            pltpu.VMEM((n_chunks, CHUNK), jnp.int32),
            pltpu.VMEM((b_per_w, EMBED), jnp.float32),
            pltpu.SemaphoreType.DMA,
        ],
    )
    def gather(tbl_hbm, idx_hbm, out_hbm, idxv, rows, sem):
        wid = lax.axis_index("s") * nc + lax.axis_index("c")
        pltpu.sync_copy(idx_hbm.at[wid], idxv)
        cps = [
            pltpu.async_copy(
                tbl_hbm.at[idxv.at[j]], rows.at[pl.ds(j * CHUNK, CHUNK)], sem
            )
            for j in range(n_chunks)
        ]
        for c in cps:
            c.wait()
        pltpu.sync_copy(rows, out_hbm.at[wid])

    return gather


def _dense_body(x0_ref, cw_ref, cb_ref, w1_ref, b1_ref, w2_ref, b2_ref,
                w3_ref, b3_ref, wox_ref, woh_ref, out_ref):
    x0 = x0_ref[...]
    x = x0
    for i in range(CROSS_LAYERS):
        xw = jnp.sum(x * cw_ref[i:i + 1, :], axis=1, keepdims=True)
        x = x0 * xw + cb_ref[i:i + 1, :] + x
    h = jnp.maximum(
        jnp.dot(x0, w1_ref[...], preferred_element_type=jnp.float32)
        + b1_ref[...], 0.0)
    h = jnp.maximum(
        jnp.dot(h, w2_ref[...], preferred_element_type=jnp.float32)
        + b2_ref[...], 0.0)
    h = jnp.maximum(
        jnp.dot(h, w3_ref[...], preferred_element_type=jnp.float32)
        + b3_ref[...], 0.0)
    out = (jnp.sum(x * wox_ref[...], axis=1)
           + jnp.sum(h * woh_ref[...], axis=1))
    out_ref[0, 0, :] = out


@functools.lru_cache(maxsize=1)
def _make_dense():
    full = lambda i: (0, 0)
    return pl.pallas_call(
        _dense_body,
        grid=(BATCH // BB,),
        in_specs=[
            pl.BlockSpec((BB, PAD_DIM), lambda i: (i, 0)),
            pl.BlockSpec((CROSS_LAYERS, PAD_DIM), full),
            pl.BlockSpec((CROSS_LAYERS, PAD_DIM), full),
            pl.BlockSpec((PAD_DIM, 512), full),
            pl.BlockSpec((1, 512), full),
            pl.BlockSpec((512, 256), full),
            pl.BlockSpec((1, 256), full),
            pl.BlockSpec((256, 128), full),
            pl.BlockSpec((1, 128), full),
            pl.BlockSpec((1, PAD_DIM), full),
            pl.BlockSpec((1, 128), full),
        ],
        out_specs=pl.BlockSpec((1, 1, BB), lambda i: (i, 0, 0)),
        out_shape=jax.ShapeDtypeStruct((BATCH // BB, 1, BB), jnp.float32),
    )


def kernel(dense, cats, tables, cross_w, cross_b, W1, b1, W2, b2, W3, b3, Wo, bo):
    pad = PAD_DIM - INPUT_DIM
    tbl_flat = tables.reshape(NUM_CAT * VOCAB, EMBED)
    offs = (jnp.arange(NUM_CAT, dtype=jnp.int32) * VOCAB)[None, :]
    flat_idx = (cats.astype(jnp.int32) + offs).reshape(32, NIDX // (32 * CHUNK), CHUNK)
    emb = _make_gather()(tbl_flat, flat_idx)
    emb_flat = emb.reshape(BATCH, NUM_CAT * EMBED)
    x0p = jnp.concatenate(
        [dense, emb_flat, jnp.zeros((BATCH, pad), jnp.float32)], axis=1)
    cw = jnp.pad(cross_w, ((0, 0), (0, pad)))
    cb = jnp.pad(cross_b, ((0, 0), (0, pad)))
    w1p = jnp.pad(W1, ((0, pad), (0, 0)))
    wox = jnp.pad(Wo[:INPUT_DIM, 0][None, :], ((0, 0), (0, pad)))
    woh = Wo[INPUT_DIM:, 0][None, :]
    out2d = _make_dense()(x0p, cw, cb, w1p, b1[None, :], W2, b2[None, :],
                          W3, b3[None, :], wox, woh)
    return out2d.reshape(BATCH) + bo[0]
